# baseline (speedup 1.0000x reference)
import functools

import jax
import jax.numpy as jnp
from jax import lax
from jax.experimental import pallas as pl
from jax.experimental.pallas import tpu as pltpu
from jax.experimental.pallas import tpu_sc as plsc

NTYPE = 7
EPS = 1e-5
NC, NS, LANES = 2, 16, 16
NW = NC * NS


def _pick_chunks(n_per):
    best = None
    for ch in (128, 112, 96, 80, 64, 48, 32, 16):
        if n_per % ch:
            continue
        nc = n_per // ch
        for d in (8, 4, 2, 1):
            if nc % d == 0:
                if best is None or (d, ch) > (best[2], best[0]):
                    best = (ch, nc, d)
                break
    assert best is not None, n_per
    return best


@functools.lru_cache(maxsize=None)
def _make_segment_sum(N, E, C):
    n_per = E // NW
    chunk, n_chunks, depth = _pick_chunks(n_per)
    n_groups = n_chunks // depth
    zrows = N // NS

    mesh = plsc.VectorSubcoreMesh(core_axis_name="c", subcore_axis_name="s")

    def body(col_h, row_h, typ_h, tab_h, out_h,
             col_v, row_v, typ_v, gidx_v, zbuf, gath, acc, se, sg, ss):
        c = lax.axis_index("c")
        s = lax.axis_index("s")
        wid = s * NC + c

        pltpu.async_copy(col_h.at[wid], col_v, se)
        pltpu.async_copy(row_h.at[wid], row_v, se)
        pltpu.async_copy(typ_h.at[wid], typ_v, se)

        zv = jnp.zeros((LANES,), jnp.float32)

        def zbody(i, carry):
            for kk in range(C // LANES):
                zbuf[i, pl.ds(kk * LANES, LANES)] = zv
            return carry
        lax.fori_loop(0, zrows, zbody, 0)
        pltpu.sync_copy(zbuf, acc.at[pl.ds(s * zrows, zrows)])

        pltpu.make_async_copy(col_h.at[wid], col_v, se).wait()
        pltpu.make_async_copy(row_h.at[wid], row_v, se).wait()
        pltpu.make_async_copy(typ_h.at[wid], typ_v, se).wait()

        def cidx_body(j, carry):
            for k in range(chunk // LANES):
                r = col_v[j, pl.ds(k * LANES, LANES)]
                t = typ_v[j, pl.ds(k * LANES, LANES)]
                gidx_v[j, pl.ds(k * LANES, LANES)] = r * NTYPE + t
            return carry
        lax.fori_loop(0, n_chunks, cidx_body, 0)

        plsc.subcore_barrier()

        for k in range(depth):
            pltpu.async_copy(tab_h.at[gidx_v.at[k]], gath[k], sg[k])

        def group_body(jj, carry):
            base = jj * depth
            for k in range(depth):
                j = base + k
                pltpu.make_async_copy(tab_h.at[gidx_v.at[0]], gath[k],
                                      sg[k]).wait()
                pltpu.async_copy(gath[k], acc.at[row_v.at[j]], ss[k], add=True)
            for k in range(depth):
                j = base + k
                pltpu.make_async_copy(gath[k], acc.at[row_v.at[0]],
                                      ss[k]).wait()
                jn = jnp.minimum(j + depth, n_chunks - 1)
                pltpu.async_copy(tab_h.at[gidx_v.at[jn]], gath[k], sg[k])
            return carry
        lax.fori_loop(0, n_groups, group_body, 0)

        for k in range(depth):
            pltpu.make_async_copy(tab_h.at[gidx_v.at[0]], gath[k], sg[k]).wait()

        plsc.subcore_barrier()

        pltpu.sync_copy(acc.at[pl.ds(s * zrows, zrows)],
                        out_h.at[c, pl.ds(s * zrows, zrows)])

    return pl.kernel(
        body,
        out_type=jax.ShapeDtypeStruct((NC, N, C), jnp.float32),
        mesh=mesh,
        scratch_types=[
            pltpu.VMEM((n_chunks, chunk), jnp.int32),
            pltpu.VMEM((n_chunks, chunk), jnp.int32),
            pltpu.VMEM((n_chunks, chunk), jnp.int32),
            pltpu.VMEM((n_chunks, chunk), jnp.int32),
            pltpu.VMEM((zrows, C), jnp.float32),
            [pltpu.VMEM((chunk, C), jnp.float32) for _ in range(depth)],
            pltpu.VMEM_SHARED((N, C), jnp.float32),
            pltpu.SemaphoreType.DMA,
            [pltpu.SemaphoreType.DMA for _ in range(depth)],
            [pltpu.SemaphoreType.DMA for _ in range(depth)],
        ],
        compiler_params=pltpu.CompilerParams(use_tc_tiling_on_sc=False),
    )


def _segment_sum(ytab, edge_idx, edge_type, N):
    C = ytab.shape[1]
    E = edge_type.shape[0]
    n_per = E // NW
    chunk, n_chunks, _ = _pick_chunks(n_per)
    col = edge_idx[1].reshape(NW, n_chunks, chunk)
    row = edge_idx[0].reshape(NW, n_chunks, chunk)
    typ = edge_type.reshape(NW, n_chunks, chunk)
    return _make_segment_sum(N, E, C)(col, row, typ, ytab)


def _premul_weights(W, Cin, Cout):
    return W.reshape(NTYPE, Cin, Cout).transpose(1, 0, 2).reshape(Cin, NTYPE * Cout)


@functools.lru_cache(maxsize=None)
def _make_premul(N, Cin, C7o, blk):
    def body(x_ref, w_ref, o_ref):
        o_ref[...] = jnp.dot(x_ref[...], w_ref[...],
                             preferred_element_type=jnp.float32)
    return pl.pallas_call(
        body,
        grid=(N // blk,),
        in_specs=[
            pl.BlockSpec((blk, Cin), lambda i: (i, 0)),
            pl.BlockSpec((Cin, C7o), lambda i: (0, 0)),
        ],
        out_specs=pl.BlockSpec((blk, C7o), lambda i: (i, 0)),
        out_shape=jax.ShapeDtypeStruct((N, C7o), jnp.float32),
    )


@functools.lru_cache(maxsize=None)
def _make_level_tail(N, C, C7o, blk):
    grid = N // blk

    def body(*refs):
        if C7o:
            p_ref, g_ref, b_ref, m_ref, v_ref, w_ref, o_ref = refs
        else:
            p_ref, g_ref, b_ref, m_ref, v_ref, o_ref = refs
        y = p_ref[0] + p_ref[1]
        scale = g_ref[...] * lax.rsqrt(v_ref[...] + EPS)
        y = scale * (y - m_ref[...]) + b_ref[...]
        y = jnp.maximum(y, 0.0)
        y = jnp.max(y.reshape(blk // 8, 8, C), axis=1)
        if C7o:
            y = jnp.dot(y, w_ref[...], preferred_element_type=jnp.float32)
        o_ref[...] = y

    bn_spec = pl.BlockSpec((1, C), lambda i: (0, 0))
    in_specs = [pl.BlockSpec((NC, blk, C), lambda i: (0, i, 0)),
                bn_spec, bn_spec, bn_spec, bn_spec]
    if C7o:
        in_specs.append(pl.BlockSpec((C, C7o), lambda i: (0, 0)))
    oc = C7o if C7o else C
    return pl.pallas_call(
        body,
        grid=(grid,),
        in_specs=in_specs,
        out_specs=pl.BlockSpec((blk // 8, oc), lambda i: (i, 0)),
        out_shape=jax.ShapeDtypeStruct((N // 8, oc), jnp.float32),
    )


def _level_tail(parts, g, b, m, v, blk, Wp=None):
    _, N, C = parts.shape
    C7o = Wp.shape[1] if Wp is not None else 0
    f = _make_level_tail(N, C, C7o, blk)
    r = lambda a: a.reshape(1, -1)
    args = (parts, r(g), r(b), r(m), r(v))
    if Wp is not None:
        args = args + (Wp,)
    return f(*args)


def _head_body(x_ref, w1_ref, g_ref, b_ref, m_ref, v_ref, w2_ref, b2_ref, o_ref):
    h = jnp.dot(x_ref[...], w1_ref[...], preferred_element_type=jnp.float32)
    scale = g_ref[...] * lax.rsqrt(v_ref[...] + EPS)
    h = scale * (h - m_ref[...]) + b_ref[...]
    h = jnp.maximum(h, 0.0)
    o_ref[...] = (jnp.dot(h, w2_ref[...], preferred_element_type=jnp.float32)
                  + b2_ref[...])


_head = pl.pallas_call(
    _head_body,
    out_shape=jax.ShapeDtypeStruct((1, 40), jnp.float32),
)


@functools.lru_cache(maxsize=None)
def _make_tail3_head():
    def body(p_ref, g_ref, b_ref, m_ref, v_ref,
             w1_ref, hg_ref, hb_ref, hm_ref, hv_ref, w2_ref, b2_ref, o_ref):
        y = p_ref[0] + p_ref[1]
        scale = g_ref[...] * lax.rsqrt(v_ref[...] + EPS)
        y = scale * (y - m_ref[...]) + b_ref[...]
        y = jnp.maximum(y, 0.0)
        y = jnp.max(y.reshape(64, 8, 64), axis=1)
        G = jnp.dot(y, w1_ref[...], preferred_element_type=jnp.float32)
        rows = lax.broadcasted_iota(jnp.int32, (64, 8192), 0)
        cols = lax.broadcasted_iota(jnp.int32, (64, 8192), 1)
        Gm = jnp.where(rows == cols // 128, G, 0.0)
        s = jnp.dot(jnp.ones((1, 64), jnp.float32), Gm,
                    preferred_element_type=jnp.float32)
        f0 = lax.broadcasted_iota(jnp.int32, (8192, 128), 0)
        f1 = lax.broadcasted_iota(jnp.int32, (8192, 128), 1)
        S = (f0 % 128 == f1).astype(jnp.float32)
        h = jnp.dot(s, S, preferred_element_type=jnp.float32)
        hscale = hg_ref[...] * lax.rsqrt(hv_ref[...] + EPS)
        h = hscale * (h - hm_ref[...]) + hb_ref[...]
        h = jnp.maximum(h, 0.0)
        o_ref[...] = (jnp.dot(h, w2_ref[...], preferred_element_type=jnp.float32)
                      + b2_ref[...])
    return pl.pallas_call(
        body,
        out_shape=jax.ShapeDtypeStruct((1, 40), jnp.float32),
    )


def kernel(data, edge_idx_5, edge_type_5, edge_idx_4, edge_type_4,
           edge_idx_3, edge_type_3,
           W5, bn5_g, bn5_b, bn5_m, bn5_v,
           W4, bn4_g, bn4_b, bn4_m, bn4_v,
           W3, bn3_g, bn3_b, bn3_m, bn3_v,
           fc1_w, fc1_g, fc1_b, fc1_m, fc1_v, fc2_w, fc2_b):
    N5, C5 = data.shape
    Co5, Co4, Co3 = W5.shape[1], W4.shape[1], W3.shape[1]

    W5p = _premul_weights(W5, C5, Co5)
    ytab = _make_premul(N5, C5, NTYPE * Co5, 4096)(data, W5p)
    parts = _segment_sum(ytab.reshape(N5 * NTYPE, Co5),
                         edge_idx_5, edge_type_5, N5)
    ytab = _level_tail(parts, bn5_g, bn5_b, bn5_m, bn5_v, blk=4096,
                       Wp=_premul_weights(W4, Co5, Co4))

    N4 = N5 // 8
    parts = _segment_sum(ytab.reshape(N4 * NTYPE, Co4),
                         edge_idx_4, edge_type_4, N4)
    ytab = _level_tail(parts, bn4_g, bn4_b, bn4_m, bn4_v, blk=4096,
                       Wp=_premul_weights(W3, Co4, Co3))

    N3 = N4 // 8
    parts = _segment_sum(ytab.reshape(N3 * NTYPE, Co3),
                         edge_idx_3, edge_type_3, N3)
    fc1_p = fc1_w.reshape(64, 8192)
    r = lambda a: a.reshape(1, -1)
    return _make_tail3_head()(parts, r(bn3_g), r(bn3_b), r(bn3_m), r(bn3_v),
                              fc1_p, r(fc1_g), r(fc1_b), r(fc1_m), r(fc1_v),
                              fc2_w, r(fc2_b))


# speedup vs baseline: 1.0951x; 1.0951x over previous
import functools

import jax
import jax.numpy as jnp
from jax import lax
from jax.experimental import pallas as pl
from jax.experimental.pallas import tpu as pltpu
from jax.experimental.pallas import tpu_sc as plsc

NTYPE = 7
EPS = 1e-5
NC, NS, LANES = 2, 16, 16
NW = NC * NS


def _pick_chunks(n_per):
    best = None
    for ch in (128, 112, 96, 80, 64, 48, 32, 16):
        if n_per % ch:
            continue
        nc = n_per // ch
        for d in (8, 4, 2, 1):
            if nc % d == 0:
                if best is None or (d, ch) > (best[2], best[0]):
                    best = (ch, nc, d)
                break
    assert best is not None, n_per
    return best


@functools.lru_cache(maxsize=None)
def _make_segment_sum(N, E, C, premul=True):
    n_per = E // NW
    chunk, n_chunks, depth = _pick_chunks(n_per)
    n_groups = n_chunks // depth
    if premul:
        n_acc = N
        half = None
        n_out = None
    else:
        half = N * NTYPE // 2
        n_acc = half + 256
        n_out = N * NTYPE
    zrows = n_acc // NS
    zparts = -(-(zrows * C * 4) // 131072)
    while zrows % zparts:
        zparts += 1
    zbrows = zrows // zparts
    orows = (half // NS) if not premul else zrows

    mesh = plsc.VectorSubcoreMesh(core_axis_name="c", subcore_axis_name="s")

    def body(col_h, row_h, typ_h, tab_h, out_h,
             col_v, row_v, typ_v, gidx_v, zbuf, gath, acc, se, sg, ss):
        c = lax.axis_index("c")
        s = lax.axis_index("s")
        wid = s * NC + c

        pltpu.async_copy(col_h.at[wid], col_v, se)
        pltpu.async_copy(row_h.at[wid], row_v, se)
        pltpu.async_copy(typ_h.at[wid], typ_v, se)

        zv = jnp.zeros((LANES,), jnp.float32)
        cw = max(1, C // LANES)

        def zbody(i, carry):
            for kk in range(cw):
                zbuf[i, pl.ds(kk * LANES, LANES)] = zv
            return carry
        lax.fori_loop(0, (zbrows * C) // (cw * LANES), zbody, 0)
        for zp in range(zparts):
            pltpu.sync_copy(zbuf,
                            acc.at[pl.ds(s * zrows + zp * zbrows, zbrows)])

        pltpu.make_async_copy(col_h.at[wid], col_v, se).wait()
        pltpu.make_async_copy(row_h.at[wid], row_v, se).wait()
        pltpu.make_async_copy(typ_h.at[wid], typ_v, se).wait()

        if premul:
            def cidx_body(j, carry):
                for k in range(chunk // LANES):
                    r = col_v[j, pl.ds(k * LANES, LANES)]
                    t = typ_v[j, pl.ds(k * LANES, LANES)]
                    gidx_v[j, pl.ds(k * LANES, LANES)] = r * NTYPE + t
                return carry
        else:
            iota16 = lax.iota(jnp.int32, LANES)

            def cidx_body(j, carry):
                toff = half + (j % 8) * LANES + iota16
                for k in range(chunk // LANES):
                    r = row_v[j, pl.ds(k * LANES, LANES)]
                    t = typ_v[j, pl.ds(k * LANES, LANES)]
                    g = r * NTYPE + t - c * half
                    owned = (g >= 0) & (g < half)
                    gidx_v[j, pl.ds(k * LANES, LANES)] = jnp.where(owned, g, toff)
                return carry
        lax.fori_loop(0, n_chunks, cidx_body, 0)

        gat_v = gidx_v if premul else col_v
        sct_v = row_v if premul else gidx_v

        plsc.subcore_barrier()

        for k in range(depth):
            pltpu.async_copy(tab_h.at[gat_v.at[k]], gath[k], sg[k])

        def group_body(jj, carry):
            base = jj * depth
            for k in range(depth):
                j = base + k
                pltpu.make_async_copy(tab_h.at[gat_v.at[0]], gath[k],
                                      sg[k]).wait()
                pltpu.async_copy(gath[k], acc.at[sct_v.at[j]], ss[k], add=True)
            for k in range(depth):
                j = base + k
                pltpu.make_async_copy(gath[k], acc.at[sct_v.at[0]],
                                      ss[k]).wait()
                jn = jnp.minimum(j + depth, n_chunks - 1)
                pltpu.async_copy(tab_h.at[gat_v.at[jn]], gath[k], sg[k])
            return carry
        lax.fori_loop(0, n_groups, group_body, 0)

        for k in range(depth):
            pltpu.make_async_copy(tab_h.at[gat_v.at[0]], gath[k], sg[k]).wait()

        plsc.subcore_barrier()

        if premul:
            pltpu.sync_copy(acc.at[pl.ds(s * zrows, zrows)],
                            out_h.at[c, pl.ds(s * zrows, zrows)])
        else:
            pltpu.sync_copy(acc.at[pl.ds(s * orows, orows)],
                            out_h.at[pl.ds(c * half + s * orows, orows)])

    return pl.kernel(
        body,
        out_type=jax.ShapeDtypeStruct(
            (NC, n_acc, C) if premul else (n_out, C), jnp.float32),
        mesh=mesh,
        scratch_types=[
            pltpu.VMEM((n_chunks, chunk), jnp.int32),
            pltpu.VMEM((n_chunks, chunk), jnp.int32),
            pltpu.VMEM((n_chunks, chunk), jnp.int32),
            pltpu.VMEM((n_chunks, chunk), jnp.int32),
            pltpu.VMEM((zbrows, C), jnp.float32),
            [pltpu.VMEM((chunk, C), jnp.float32) for _ in range(depth)],
            pltpu.VMEM_SHARED((n_acc, C), jnp.float32),
            pltpu.SemaphoreType.DMA,
            [pltpu.SemaphoreType.DMA for _ in range(depth)],
            [pltpu.SemaphoreType.DMA for _ in range(depth)],
        ],
        compiler_params=pltpu.CompilerParams(use_tc_tiling_on_sc=False),
    )


def _segment_sum(tab, edge_idx, edge_type, N, premul=True):
    C = tab.shape[1]
    E = edge_type.shape[0]
    n_per = E // NW
    chunk, n_chunks, _ = _pick_chunks(n_per)
    col = edge_idx[1].reshape(NW, n_chunks, chunk)
    row = edge_idx[0].reshape(NW, n_chunks, chunk)
    typ = edge_type.reshape(NW, n_chunks, chunk)
    return _make_segment_sum(N, E, C, premul)(col, row, typ, tab)


def _premul_weights(W, Cin, Cout):
    return W.reshape(NTYPE, Cin, Cout).transpose(1, 0, 2).reshape(Cin, NTYPE * Cout)


@functools.lru_cache(maxsize=None)
def _make_premul(N, Cin, C7o, blk):
    def body(x_ref, w_ref, o_ref):
        o_ref[...] = jnp.dot(x_ref[...], w_ref[...],
                             preferred_element_type=jnp.float32)
    return pl.pallas_call(
        body,
        grid=(N // blk,),
        in_specs=[
            pl.BlockSpec((blk, Cin), lambda i: (i, 0)),
            pl.BlockSpec((Cin, C7o), lambda i: (0, 0)),
        ],
        out_specs=pl.BlockSpec((blk, C7o), lambda i: (i, 0)),
        out_shape=jax.ShapeDtypeStruct((N, C7o), jnp.float32),
    )


@functools.lru_cache(maxsize=None)
def _make_level_tail(N, C, C7o, blk):
    grid = N // blk

    def body(*refs):
        if C7o:
            p_ref, g_ref, b_ref, m_ref, v_ref, w_ref, o_ref = refs
        else:
            p_ref, g_ref, b_ref, m_ref, v_ref, o_ref = refs
        y = p_ref[0] + p_ref[1]
        scale = g_ref[...] * lax.rsqrt(v_ref[...] + EPS)
        y = scale * (y - m_ref[...]) + b_ref[...]
        y = jnp.maximum(y, 0.0)
        y = jnp.max(y.reshape(blk // 8, 8, C), axis=1)
        if C7o:
            y = jnp.dot(y, w_ref[...], preferred_element_type=jnp.float32)
        o_ref[...] = y

    bn_spec = pl.BlockSpec((1, C), lambda i: (0, 0))
    in_specs = [pl.BlockSpec((NC, blk, C), lambda i: (0, i, 0)),
                bn_spec, bn_spec, bn_spec, bn_spec]
    if C7o:
        in_specs.append(pl.BlockSpec((C, C7o), lambda i: (0, 0)))
    oc = C7o if C7o else C
    return pl.pallas_call(
        body,
        grid=(grid,),
        in_specs=in_specs,
        out_specs=pl.BlockSpec((blk // 8, oc), lambda i: (i, 0)),
        out_shape=jax.ShapeDtypeStruct((N // 8, oc), jnp.float32),
    )


def _level_tail(parts, g, b, m, v, blk, Wp=None):
    _, N, C = parts.shape
    C7o = Wp.shape[1] if Wp is not None else 0
    f = _make_level_tail(N, C, C7o, blk)
    r = lambda a: a.reshape(1, -1)
    args = (parts, r(g), r(b), r(m), r(v))
    if Wp is not None:
        args = args + (Wp,)
    return f(*args)


@functools.lru_cache(maxsize=None)
def _make_level_tail5(N, C7, Cmid, C7o, blk):
    def body(p_ref, w_ref, g_ref, b_ref, m_ref, v_ref, wp_ref, o_ref):
        y = jnp.dot(p_ref[...], w_ref[...], preferred_element_type=jnp.float32)
        scale = g_ref[...] * lax.rsqrt(v_ref[...] + EPS)
        y = scale * (y - m_ref[...]) + b_ref[...]
        y = jnp.maximum(y, 0.0)
        y = jnp.max(y.reshape(blk // 8, 8, Cmid), axis=1)
        o_ref[...] = jnp.dot(y, wp_ref[...], preferred_element_type=jnp.float32)

    bn_spec = pl.BlockSpec((1, Cmid), lambda i: (0, 0))
    return pl.pallas_call(
        body,
        grid=(N // blk,),
        in_specs=[pl.BlockSpec((blk, C7), lambda i: (i, 0)),
                  pl.BlockSpec((C7, Cmid), lambda i: (0, 0)),
                  bn_spec, bn_spec, bn_spec, bn_spec,
                  pl.BlockSpec((Cmid, C7o), lambda i: (0, 0))],
        out_specs=pl.BlockSpec((blk // 8, C7o), lambda i: (i, 0)),
        out_shape=jax.ShapeDtypeStruct((N // 8, C7o), jnp.float32),
    )


def _level_tail5(parts, W, g, b, m, v, Wp):
    N, C7 = parts.shape
    f = _make_level_tail5(N, C7, W.shape[1], Wp.shape[1], 4096)
    r = lambda a: a.reshape(1, -1)
    return f(parts, W, r(g), r(b), r(m), r(v), Wp)


def _head_body(x_ref, w1_ref, g_ref, b_ref, m_ref, v_ref, w2_ref, b2_ref, o_ref):
    h = jnp.dot(x_ref[...], w1_ref[...], preferred_element_type=jnp.float32)
    scale = g_ref[...] * lax.rsqrt(v_ref[...] + EPS)
    h = scale * (h - m_ref[...]) + b_ref[...]
    h = jnp.maximum(h, 0.0)
    o_ref[...] = (jnp.dot(h, w2_ref[...], preferred_element_type=jnp.float32)
                  + b2_ref[...])


_head = pl.pallas_call(
    _head_body,
    out_shape=jax.ShapeDtypeStruct((1, 40), jnp.float32),
)


@functools.lru_cache(maxsize=None)
def _make_tail3_head():
    def body(p_ref, g_ref, b_ref, m_ref, v_ref,
             w1_ref, hg_ref, hb_ref, hm_ref, hv_ref, w2_ref, b2_ref, o_ref):
        y = p_ref[0] + p_ref[1]
        scale = g_ref[...] * lax.rsqrt(v_ref[...] + EPS)
        y = scale * (y - m_ref[...]) + b_ref[...]
        y = jnp.maximum(y, 0.0)
        y = jnp.max(y.reshape(64, 8, 64), axis=1)
        G = jnp.dot(y, w1_ref[...], preferred_element_type=jnp.float32)
        rows = lax.broadcasted_iota(jnp.int32, (64, 8192), 0)
        cols = lax.broadcasted_iota(jnp.int32, (64, 8192), 1)
        Gm = jnp.where(rows == cols // 128, G, 0.0)
        s = jnp.dot(jnp.ones((1, 64), jnp.float32), Gm,
                    preferred_element_type=jnp.float32)
        f0 = lax.broadcasted_iota(jnp.int32, (8192, 128), 0)
        f1 = lax.broadcasted_iota(jnp.int32, (8192, 128), 1)
        S = (f0 % 128 == f1).astype(jnp.float32)
        h = jnp.dot(s, S, preferred_element_type=jnp.float32)
        hscale = hg_ref[...] * lax.rsqrt(hv_ref[...] + EPS)
        h = hscale * (h - hm_ref[...]) + hb_ref[...]
        h = jnp.maximum(h, 0.0)
        o_ref[...] = (jnp.dot(h, w2_ref[...], preferred_element_type=jnp.float32)
                      + b2_ref[...])
    return pl.pallas_call(
        body,
        out_shape=jax.ShapeDtypeStruct((1, 40), jnp.float32),
    )


def kernel(data, edge_idx_5, edge_type_5, edge_idx_4, edge_type_4,
           edge_idx_3, edge_type_3,
           W5, bn5_g, bn5_b, bn5_m, bn5_v,
           W4, bn4_g, bn4_b, bn4_m, bn4_v,
           W3, bn3_g, bn3_b, bn3_m, bn3_v,
           fc1_w, fc1_g, fc1_b, fc1_m, fc1_v, fc2_w, fc2_b):
    N5, C5 = data.shape
    Co5, Co4, Co3 = W5.shape[1], W4.shape[1], W3.shape[1]

    Cp = 8
    x_pad = jnp.pad(data, ((0, 0), (0, Cp - C5)))
    seg = _segment_sum(x_pad, edge_idx_5, edge_type_5, N5, premul=False)
    seg = seg.reshape(N5, NTYPE * Cp)
    W5pad = jnp.pad(W5.reshape(NTYPE, C5, Co5),
                    ((0, 0), (0, Cp - C5), (0, 0))).reshape(NTYPE * Cp, Co5)
    ytab = _level_tail5(seg, W5pad, bn5_g, bn5_b, bn5_m, bn5_v,
                        _premul_weights(W4, Co5, Co4))

    N4 = N5 // 8
    parts = _segment_sum(ytab.reshape(N4 * NTYPE, Co4),
                         edge_idx_4, edge_type_4, N4)
    ytab = _level_tail(parts, bn4_g, bn4_b, bn4_m, bn4_v, blk=4096,
                       Wp=_premul_weights(W3, Co4, Co3))

    N3 = N4 // 8
    parts = _segment_sum(ytab.reshape(N3 * NTYPE, Co3),
                         edge_idx_3, edge_type_3, N3)
    fc1_p = fc1_w.reshape(64, 8192)
    r = lambda a: a.reshape(1, -1)
    return _make_tail3_head()(parts, r(bn3_g), r(bn3_b), r(bn3_m), r(bn3_v),
                              fc1_p, r(fc1_g), r(fc1_b), r(fc1_m), r(fc1_v),
                              fc2_w, r(fc2_b))
